# Initial kernel scaffold; baseline (speedup 1.0000x reference)
import jax
import jax.numpy as jnp
from jax.experimental import pallas as pl


def kernel(prefix, embedding_weight):
    raise NotImplementedError("write your pallas kernel here")


# speedup vs baseline: 1.1818x; 1.1818x over previous
import functools

import jax
import jax.numpy as jnp
from jax import lax
from jax.experimental import pallas as pl
from jax.experimental.pallas import tpu as pltpu
from jax.experimental.pallas import tpu_sc as plsc

_NUM_CORES = 2
_NUM_SUBCORES = 16
_NW = _NUM_CORES * _NUM_SUBCORES
_CHUNK = 16


def kernel(prefix, embedding_weight):
    batch, seq = prefix.shape
    vocab, row_dim = embedding_weight.shape
    n = batch * seq
    bpw = n // _NW
    idx = prefix.reshape(n)

    mesh = plsc.VectorSubcoreMesh(core_axis_name="c", subcore_axis_name="s")

    @functools.partial(
        pl.kernel,
        mesh=mesh,
        out_type=jax.ShapeDtypeStruct((n, row_dim), jnp.float32),
        scratch_types=[
            pltpu.VMEM((bpw,), jnp.int32),
            pltpu.VMEM((_CHUNK, row_dim), jnp.float32),
        ],
    )
    def sc_gather(table_hbm, idx_hbm, out_hbm, idx_v, rows_v):
        wid = lax.axis_index("s") * _NUM_CORES + lax.axis_index("c")
        base = wid * bpw
        pltpu.sync_copy(idx_hbm.at[pl.ds(base, bpw)], idx_v)

        @pl.loop(0, bpw, step=_CHUNK)
        def _(off):
            pltpu.sync_copy(table_hbm.at[idx_v.at[pl.ds(off, _CHUNK)]], rows_v)
            pltpu.sync_copy(rows_v, out_hbm.at[pl.ds(base + off, _CHUNK)])

    out = sc_gather(embedding_weight, idx)
    return out.reshape(batch, seq, row_dim)


# speedup vs baseline: 1.2107x; 1.0245x over previous
import functools

import jax
import jax.numpy as jnp
from jax import lax
from jax.experimental import pallas as pl
from jax.experimental.pallas import tpu as pltpu
from jax.experimental.pallas import tpu_sc as plsc

_NUM_CORES = 2
_NUM_SUBCORES = 16
_NW = _NUM_CORES * _NUM_SUBCORES
_CHUNK = 16
_NBUF = 2


def kernel(prefix, embedding_weight):
    batch, seq = prefix.shape
    vocab, row_dim = embedding_weight.shape
    n = batch * seq
    bpw = n // _NW
    step = _NBUF * _CHUNK
    idx = prefix.reshape(n)

    mesh = plsc.VectorSubcoreMesh(core_axis_name="c", subcore_axis_name="s")

    @functools.partial(
        pl.kernel,
        mesh=mesh,
        out_type=jax.ShapeDtypeStruct((n, row_dim), jnp.float32),
        scratch_types=[
            pltpu.VMEM((bpw,), jnp.int32),
            pltpu.VMEM((_CHUNK, row_dim), jnp.float32),
            pltpu.VMEM((_CHUNK, row_dim), jnp.float32),
            pltpu.SemaphoreType.DMA,
            pltpu.SemaphoreType.DMA,
            pltpu.SemaphoreType.DMA,
            pltpu.SemaphoreType.DMA,
        ],
    )
    def sc_gather(table_hbm, idx_hbm, out_hbm, idx_v, rows0, rows1,
                  gsem0, gsem1, wsem0, wsem1):
        wid = lax.axis_index("s") * _NUM_CORES + lax.axis_index("c")
        base = wid * bpw
        pltpu.sync_copy(idx_hbm.at[pl.ds(base, bpw)], idx_v)

        bufs = (rows0, rows1)
        gsems = (gsem0, gsem1)
        wsems = (wsem0, wsem1)

        def g_copy(off, b):
            return pltpu.make_async_copy(
                table_hbm.at[idx_v.at[pl.ds(off, _CHUNK)]], bufs[b], gsems[b])

        def w_copy(off, b):
            return pltpu.make_async_copy(
                bufs[b], out_hbm.at[pl.ds(base + off, _CHUNK)], wsems[b])

        for b in range(_NBUF):
            g_copy(b * _CHUNK, b).start()

        @pl.loop(0, bpw - step, step=step)
        def _(off):
            for b in range(_NBUF):
                o = off + b * _CHUNK
                g_copy(o, b).wait()
                w_copy(o, b).start()
            for b in range(_NBUF):
                o = off + b * _CHUNK
                w_copy(o, b).wait()
                g_copy(o + step, b).start()

        off_last = bpw - step
        for b in range(_NBUF):
            o = off_last + b * _CHUNK
            g_copy(o, b).wait()
            w_copy(o, b).start()
        for b in range(_NBUF):
            o = off_last + b * _CHUNK
            w_copy(o, b).wait()

    out = sc_gather(embedding_weight, idx)
    return out.reshape(batch, seq, row_dim)


# speedup vs baseline: 1.7804x; 1.4705x over previous
import functools

import jax
import jax.numpy as jnp
from jax import lax
from jax.experimental import pallas as pl
from jax.experimental.pallas import tpu as pltpu
from jax.experimental.pallas import tpu_sc as plsc

_NUM_CORES = 2
_NUM_SUBCORES = 16
_NW = _NUM_CORES * _NUM_SUBCORES


def kernel(prefix, embedding_weight):
    batch, seq = prefix.shape
    vocab, row_dim = embedding_weight.shape
    half = row_dim // 2
    bpw = batch // _NW

    table_lo = embedding_weight[:, :half]
    table_hi = embedding_weight[:, half:]

    mesh = plsc.VectorSubcoreMesh(core_axis_name="c", subcore_axis_name="s")

    @functools.partial(
        pl.kernel,
        mesh=mesh,
        out_type=jax.ShapeDtypeStruct((batch, seq, row_dim), jnp.float32),
        scratch_types=[
            pltpu.VMEM((bpw, seq), jnp.int32),
            pltpu.VMEM((seq, half), jnp.float32),
            pltpu.VMEM((seq, half), jnp.float32),
            pltpu.SemaphoreType.DMA,
            pltpu.SemaphoreType.DMA,
            pltpu.SemaphoreType.DMA,
            pltpu.SemaphoreType.DMA,
        ],
    )
    def sc_gather(lo_hbm, hi_hbm, idx_hbm, out_hbm, idx_v, rows0, rows1,
                  gsem0, gsem1, wsem0, wsem1):
        wid = lax.axis_index("s") * _NUM_CORES + lax.axis_index("c")
        base = wid * bpw
        pltpu.sync_copy(idx_hbm.at[pl.ds(base, bpw)], idx_v)

        tables = (lo_hbm, hi_hbm)
        bufs = (rows0, rows1)
        gsems = (gsem0, gsem1)
        wsems = (wsem0, wsem1)

        def g_copy(k, b):
            return pltpu.make_async_copy(
                tables[b].at[idx_v.at[k]], bufs[b], gsems[b])

        def w_copy(k, b):
            return pltpu.make_async_copy(
                bufs[b], out_hbm.at[base + k, :, pl.ds(b * half, half)],
                wsems[b])

        for b in range(2):
            g_copy(0, b).start()

        @pl.loop(0, bpw - 1)
        def _(k):
            for b in range(2):
                g_copy(k, b).wait()
                w_copy(k, b).start()
            for b in range(2):
                w_copy(k, b).wait()
                g_copy(k + 1, b).start()

        k_last = bpw - 1
        for b in range(2):
            g_copy(k_last, b).wait()
            w_copy(k_last, b).start()
        for b in range(2):
            w_copy(k_last, b).wait()

    return sc_gather(table_lo, table_hi, prefix)


# speedup vs baseline: 3.7182x; 2.0884x over previous
import functools

import jax
import jax.numpy as jnp
from jax import lax
from jax.experimental import pallas as pl
from jax.experimental.pallas import tpu as pltpu
from jax.experimental.pallas import tpu_sc as plsc

_NUM_CORES = 2
_NUM_SUBCORES = 16
_NW = _NUM_CORES * _NUM_SUBCORES
_BLK = 16


def kernel(prefix, embedding_weight):
    batch, seq = prefix.shape
    vocab, row_dim = embedding_weight.shape
    bpw = batch // _NW
    nh = bpw // _BLK

    idx_t = prefix.T

    mesh = plsc.VectorSubcoreMesh(core_axis_name="c", subcore_axis_name="s")

    @functools.partial(
        pl.kernel,
        mesh=mesh,
        out_type=jax.ShapeDtypeStruct((seq, batch, row_dim), jnp.float32),
        scratch_types=[
            pltpu.VMEM((seq, 128), jnp.int32),
            pltpu.VMEM((_BLK, row_dim), jnp.float32),
            pltpu.VMEM((_BLK, row_dim), jnp.float32),
            pltpu.SemaphoreType.DMA,
            pltpu.SemaphoreType.DMA,
            pltpu.SemaphoreType.DMA,
            pltpu.SemaphoreType.DMA,
        ],
    )
    def sc_gather(table_hbm, idx_hbm, out_hbm, idx_v, rows0, rows1,
                  gsem0, gsem1, wsem0, wsem1):
        wid = lax.axis_index("s") * _NUM_CORES + lax.axis_index("c")
        base = wid * bpw
        pltpu.sync_copy(idx_hbm.at[:, pl.ds((wid // 4) * 128, 128)], idx_v)
        col0 = (wid % 4) * bpw

        bufs = (rows0, rows1)
        gsems = (gsem0, gsem1)
        wsems = (wsem0, wsem1)

        def g_copy(s, b):
            return pltpu.make_async_copy(
                table_hbm.at[idx_v.at[s, pl.ds(col0 + b * _BLK, _BLK)]],
                bufs[b], gsems[b])

        def w_copy(s, b):
            return pltpu.make_async_copy(
                bufs[b], out_hbm.at[s, pl.ds(base + b * _BLK, _BLK)],
                wsems[b])

        for b in range(nh):
            g_copy(0, b).start()

        @pl.loop(0, seq - 1)
        def _(s):
            for b in range(nh):
                g_copy(s, b).wait()
                w_copy(s, b).start()
            for b in range(nh):
                w_copy(s, b).wait()
                g_copy(s + 1, b).start()

        s_last = seq - 1
        for b in range(nh):
            g_copy(s_last, b).wait()
            w_copy(s_last, b).start()
        for b in range(nh):
            w_copy(s_last, b).wait()

    out_t = sc_gather(embedding_weight, idx_t)
    return out_t.transpose(1, 0, 2)


# speedup vs baseline: 3.7799x; 1.0166x over previous
import functools

import jax
import jax.numpy as jnp
from jax import lax
from jax.experimental import pallas as pl
from jax.experimental.pallas import tpu as pltpu
from jax.experimental.pallas import tpu_sc as plsc

_NUM_CORES = 2
_NUM_SUBCORES = 16
_NW = _NUM_CORES * _NUM_SUBCORES
_BLK = 16
_NBUF = 2
_REP = 8


def kernel(prefix, embedding_weight):
    batch, seq = prefix.shape
    vocab, row_dim = embedding_weight.shape
    bpw = batch // _NW
    nh = bpw // _BLK

    table_rep = jnp.tile(embedding_weight, (_REP, 1))
    rep_off = ((jnp.arange(batch, dtype=jnp.int32) // bpw) % _REP) * vocab
    idx_t = prefix.T + rep_off[None, :]

    mesh = plsc.VectorSubcoreMesh(core_axis_name="c", subcore_axis_name="s")

    @functools.partial(
        pl.kernel,
        mesh=mesh,
        out_type=jax.ShapeDtypeStruct((seq, batch, row_dim), jnp.float32),
        scratch_types=[
            pltpu.VMEM((seq, 128), jnp.int32),
            pltpu.VMEM((_BLK, row_dim), jnp.float32),
            pltpu.VMEM((_BLK, row_dim), jnp.float32),
            pltpu.SemaphoreType.DMA,
            pltpu.SemaphoreType.DMA,
            pltpu.SemaphoreType.DMA,
            pltpu.SemaphoreType.DMA,
        ],
    )
    def sc_gather(table_hbm, idx_hbm, out_hbm, idx_v, rows0, rows1,
                  gsem0, gsem1, wsem0, wsem1):
        wid = lax.axis_index("s") * _NUM_CORES + lax.axis_index("c")
        base = wid * bpw
        pltpu.sync_copy(idx_hbm.at[:, pl.ds((wid // 4) * 128, 128)], idx_v)
        col0 = (wid % 4) * bpw

        bufs = (rows0, rows1)
        gsems = (gsem0, gsem1)
        wsems = (wsem0, wsem1)

        n_items = seq * nh

        def g_copy(t, b):
            s, q = t // nh, t % nh
            return pltpu.make_async_copy(
                table_hbm.at[idx_v.at[s, pl.ds(col0 + q * _BLK, _BLK)]],
                bufs[b], gsems[b])

        def w_copy(t, b):
            s, q = t // nh, t % nh
            return pltpu.make_async_copy(
                bufs[b], out_hbm.at[s, pl.ds(base + q * _BLK, _BLK)],
                wsems[b])

        for b in range(_NBUF):
            g_copy(b, b).start()

        @pl.loop(0, n_items - _NBUF, step=_NBUF)
        def _(t):
            for b in range(_NBUF):
                g_copy(t + b, b).wait()
                w_copy(t + b, b).start()
            for b in range(_NBUF):
                w_copy(t + b, b).wait()
                g_copy(t + b + _NBUF, b).start()

        t_last = n_items - _NBUF
        for b in range(_NBUF):
            g_copy(t_last + b, b).wait()
            w_copy(t_last + b, b).start()
        for b in range(_NBUF):
            w_copy(t_last + b, b).wait()

    out_t = sc_gather(table_rep, idx_t)
    return out_t.transpose(1, 0, 2)


# speedup vs baseline: 3.7982x; 1.0048x over previous
import functools

import jax
import jax.numpy as jnp
from jax import lax
from jax.experimental import pallas as pl
from jax.experimental.pallas import tpu as pltpu
from jax.experimental.pallas import tpu_sc as plsc

_NUM_CORES = 2
_NUM_SUBCORES = 16
_NW = _NUM_CORES * _NUM_SUBCORES
_BLK = 8
_NBUF = 4
_REP = 8


def kernel(prefix, embedding_weight):
    batch, seq = prefix.shape
    vocab, row_dim = embedding_weight.shape
    bpw = batch // _NW
    nh = bpw // _BLK

    table_rep = jnp.tile(embedding_weight, (_REP, 1))
    rep_off = ((jnp.arange(batch, dtype=jnp.int32) // bpw) % _REP) * vocab
    idx_t = prefix.T + rep_off[None, :]

    mesh = plsc.VectorSubcoreMesh(core_axis_name="c", subcore_axis_name="s")

    @functools.partial(
        pl.kernel,
        mesh=mesh,
        out_type=jax.ShapeDtypeStruct((seq, batch, row_dim), jnp.float32),
        scratch_types=[
            pltpu.VMEM((seq, 128), jnp.int32),
            pltpu.VMEM((_BLK, row_dim), jnp.float32),
            pltpu.VMEM((_BLK, row_dim), jnp.float32),
            pltpu.VMEM((_BLK, row_dim), jnp.float32),
            pltpu.VMEM((_BLK, row_dim), jnp.float32),
        ] + [pltpu.SemaphoreType.DMA] * 8,
    )
    def sc_gather(table_hbm, idx_hbm, out_hbm, idx_v, rows0, rows1,
                  rows2, rows3, gsem0, gsem1, gsem2, gsem3,
                  wsem0, wsem1, wsem2, wsem3):
        wid = lax.axis_index("s") * _NUM_CORES + lax.axis_index("c")
        base = wid * bpw
        pltpu.sync_copy(idx_hbm.at[:, pl.ds((wid // 4) * 128, 128)], idx_v)
        col0 = (wid % 4) * bpw

        bufs = (rows0, rows1, rows2, rows3)
        gsems = (gsem0, gsem1, gsem2, gsem3)
        wsems = (wsem0, wsem1, wsem2, wsem3)

        n_items = seq * nh

        def g_copy(t, b):
            s, q = t // nh, t % nh
            return pltpu.make_async_copy(
                table_hbm.at[idx_v.at[s, pl.ds(col0 + q * _BLK, _BLK)]],
                bufs[b], gsems[b])

        def w_copy(t, b):
            s, q = t // nh, t % nh
            return pltpu.make_async_copy(
                bufs[b], out_hbm.at[s, pl.ds(base + q * _BLK, _BLK)],
                wsems[b])

        for b in range(_NBUF):
            g_copy(b, b).start()

        @pl.loop(0, n_items - _NBUF, step=_NBUF)
        def _(t):
            for b in range(_NBUF):
                g_copy(t + b, b).wait()
                w_copy(t + b, b).start()
            for b in range(_NBUF):
                w_copy(t + b, b).wait()
                g_copy(t + b + _NBUF, b).start()

        t_last = n_items - _NBUF
        for b in range(_NBUF):
            g_copy(t_last + b, b).wait()
            w_copy(t_last + b, b).start()
        for b in range(_NBUF):
            w_copy(t_last + b, b).wait()

    out_t = sc_gather(table_rep, idx_t)
    return out_t.transpose(1, 0, 2)
